# baseline (speedup 1.0000x reference)
import functools

import jax
import jax.numpy as jnp
from jax import lax
from jax.experimental import pallas as pl
from jax.experimental.pallas import tpu as pltpu
from jax.experimental.pallas import tpu_sc as plsc

_EPS = 1e-5
_NC, _NS = 2, 16
_NW = _NC * _NS


def _ln(y, g, b):
    mean = jnp.mean(y, axis=1, keepdims=True)
    yc = y - mean
    var = jnp.mean(yc * yc, axis=1, keepdims=True)
    return yc * jax.lax.rsqrt(var + _EPS) * g + b


def _fused_body(ids_ref, x_ref, tbl_ref, g_ref, b_ref, o_ref):
    ids = ids_ref[0, 0, :]
    r = ids.shape[0]
    k = tbl_ref.shape[0]
    onehot = (ids[:, None] == jax.lax.broadcasted_iota(jnp.int32, (r, k), 1))
    e = jnp.dot(onehot.astype(jnp.float32), tbl_ref[...],
                preferred_element_type=jnp.float32)
    o_ref[...] = _ln(x_ref[...] + e, g_ref[...], b_ref[...])


def _addln_body(x_ref, e_ref, g_ref, b_ref, o_ref):
    o_ref[...] = _ln(x_ref[...] + e_ref[...], g_ref[...], b_ref[...])


def _sc_gather(q, h, chunk=64):
    b_per_w = q // _NW
    n_chunks = b_per_w // chunk
    mesh = plsc.VectorSubcoreMesh(core_axis_name="c", subcore_axis_name="s")

    @functools.partial(
        pl.kernel, mesh=mesh,
        out_type=jax.ShapeDtypeStruct((q, h), jnp.float32),
        scratch_types=[
            pltpu.VMEM((chunk,), jnp.int32),
            pltpu.VMEM((chunk, h), jnp.float32),
            pltpu.SemaphoreType.DMA,
        ],
    )
    def k(table_hbm, idx_hbm, out_hbm, idx_v, rows_v, sem):
        wid = lax.axis_index("s") * _NC + lax.axis_index("c")
        for c in range(n_chunks):
            base = wid * b_per_w + c * chunk
            pltpu.sync_copy(idx_hbm.at[pl.ds(base, chunk)], idx_v)
            pltpu.async_copy(table_hbm.at[idx_v], rows_v, sem).wait()
            pltpu.sync_copy(rows_v, out_hbm.at[pl.ds(base, chunk)])

    return k


def _tc_fused(x, ids, tbl, g, b, r=2048):
    n, h = x.shape
    nblk = n // r
    k = tbl.shape[0]
    ids3 = ids.reshape(nblk, 1, r)
    return pl.pallas_call(
        _fused_body,
        grid=(nblk,),
        in_specs=[
            pl.BlockSpec((1, 1, r), lambda i: (i, 0, 0)),
            pl.BlockSpec((r, h), lambda i: (i, 0)),
            pl.BlockSpec((k, h), lambda i: (0, 0)),
            pl.BlockSpec((1, h), lambda i: (0, 0)),
            pl.BlockSpec((1, h), lambda i: (0, 0)),
        ],
        out_specs=pl.BlockSpec((r, h), lambda i: (i, 0)),
        out_shape=jax.ShapeDtypeStruct((n, h), jnp.float32),
        compiler_params=pltpu.CompilerParams(dimension_semantics=("arbitrary",)),
    )(ids3, x, tbl, g, b)


def _tc_addln(x, e, g, b, r=2048):
    n, h = x.shape
    nblk = n // r
    return pl.pallas_call(
        _addln_body,
        grid=(nblk,),
        in_specs=[
            pl.BlockSpec((r, h), lambda i: (i, 0)),
            pl.BlockSpec((r, h), lambda i: (i, 0)),
            pl.BlockSpec((1, h), lambda i: (0, 0)),
            pl.BlockSpec((1, h), lambda i: (0, 0)),
        ],
        out_specs=pl.BlockSpec((r, h), lambda i: (i, 0)),
        out_shape=jax.ShapeDtypeStruct((n, h), jnp.float32),
        compiler_params=pltpu.CompilerParams(dimension_semantics=("arbitrary",)),
    )(x, e, g, b)


def kernel(batch_mention_emb, mention_type_ids, emb_table, ln_gamma, ln_beta):
    b, s, h = batch_mention_emb.shape
    ids = mention_type_ids.astype(jnp.int32)
    g = ln_gamma.reshape(1, h)
    bb = ln_beta.reshape(1, h)
    ktab = 128
    tbl = jnp.zeros((ktab, h), emb_table.dtype).at[: emb_table.shape[0]].set(emb_table)

    q = s
    e0 = _sc_gather(q, h)(emb_table, ids[0])
    out0 = _tc_addln(batch_mention_emb[0], e0, g, bb)

    n_rest = (b - 1) * s
    x_rest = batch_mention_emb[1:].reshape(n_rest, h)
    out_rest = _tc_fused(x_rest, ids[1:].reshape(n_rest), tbl, g, bb)

    out = jnp.concatenate([out0[None], out_rest.reshape(b - 1, s, h)], axis=0)
    return out


# speedup vs baseline: 3.5407x; 3.5407x over previous
import jax
import jax.numpy as jnp
from jax.experimental import pallas as pl
from jax.experimental.pallas import tpu as pltpu

_EPS = 1e-5


def _fused_body(ids_ref, x_ref, tbl_ref, g_ref, b_ref, o_ref):
    ids = ids_ref[0, 0, :]
    r = ids.shape[0]
    k = tbl_ref.shape[0]
    onehot = (ids[:, None] == jax.lax.broadcasted_iota(jnp.int32, (r, k), 1))
    e = jnp.dot(onehot.astype(jnp.float32), tbl_ref[...],
                preferred_element_type=jnp.float32)
    y = x_ref[...] + e
    mean = jnp.mean(y, axis=1, keepdims=True)
    yc = y - mean
    var = jnp.mean(yc * yc, axis=1, keepdims=True)
    o_ref[...] = yc * jax.lax.rsqrt(var + _EPS) * g_ref[...] + b_ref[...]


def kernel(batch_mention_emb, mention_type_ids, emb_table, ln_gamma, ln_beta):
    b, s, h = batch_mention_emb.shape
    n = b * s
    r = 2048
    nblk = n // r
    x = batch_mention_emb.reshape(n, h)
    ids = mention_type_ids.reshape(nblk, 1, r).astype(jnp.int32)
    k = 128
    tbl = jnp.zeros((k, h), emb_table.dtype).at[: emb_table.shape[0]].set(emb_table)
    out = pl.pallas_call(
        _fused_body,
        grid=(nblk,),
        in_specs=[
            pl.BlockSpec((1, 1, r), lambda i: (i, 0, 0)),
            pl.BlockSpec((r, h), lambda i: (i, 0)),
            pl.BlockSpec((k, h), lambda i: (0, 0)),
            pl.BlockSpec((1, h), lambda i: (0, 0)),
            pl.BlockSpec((1, h), lambda i: (0, 0)),
        ],
        out_specs=pl.BlockSpec((r, h), lambda i: (i, 0)),
        out_shape=jax.ShapeDtypeStruct((n, h), jnp.float32),
        compiler_params=pltpu.CompilerParams(dimension_semantics=("parallel",)),
    )(ids, x, tbl, ln_gamma.reshape(1, h), ln_beta.reshape(1, h))
    return out.reshape(b, s, h)
